# baseline (speedup 1.0000x reference)
import functools

import jax
import jax.numpy as jnp
from jax.experimental import pallas as pl
from jax.experimental.pallas import tpu as pltpu

INTERP = False

F32 = jnp.float32
BF16 = jnp.bfloat16
DIMS_NT = (((1,), (1,)), ((), ()))


def _dx(a, b):
    return jnp.dot(a, b, precision=jax.lax.Precision.HIGHEST)


def _dot(a, b):
    return jnp.dot(a.astype(BF16), b.astype(BF16), preferred_element_type=F32)


def _ln(x, g, b, eps=1e-5):
    m = jnp.mean(x, axis=-1, keepdims=True)
    v = jnp.mean((x - m) ** 2, axis=-1, keepdims=True)
    return (x - m) * jax.lax.rsqrt(v + eps) * g + b


def _eye(n):
    ii = jax.lax.broadcasted_iota(jnp.int32, (n, n), 0)
    jj = jax.lax.broadcasted_iota(jnp.int32, (n, n), 1)
    return (ii == jj).astype(F32)


def _to_row(col, n):
    bm = col * jnp.ones((n, n), F32)
    return _dx(jnp.ones((1, n), F32), _eye(n) * bm)


def _to_col(row, n):
    am = _dx(jnp.ones((n, 1), F32), row)
    return _dx(_eye(n) * am, jnp.ones((n, 1), F32))


def _topk_from_col(s_col, n, k):
    ones_nn = jnp.ones((n, n), F32)
    bm = s_col * ones_nn
    am = _dx(ones_nn, _eye(n) * bm)
    ii = jax.lax.broadcasted_iota(jnp.int32, (n, n), 0)
    jj = jax.lax.broadcasted_iota(jnp.int32, (n, n), 1)
    gt = (am > bm).astype(F32)
    tie = ((am == bm) & (jj < ii)).astype(F32)
    rank_col = jnp.sum(gt + tie, axis=1, keepdims=True)
    rank_row = _to_row(rank_col, n)
    rr = rank_row + jnp.zeros((k, n), F32)
    kio = jax.lax.broadcasted_iota(jnp.int32, (k, n), 0).astype(F32)
    sel = (rr == kio).astype(F32)
    j_col = jax.lax.broadcasted_iota(jnp.int32, (n, 1), 0).astype(F32)
    idx_col = _dx(sel, j_col)
    vals_col = _dx(sel, s_col)
    return sel, vals_col, idx_col


def _gelu(x):
    return 0.5 * x * (1.0 + jax.lax.erf(x * 0.7071067811865476))


def _softmax_col(v):
    m = jnp.max(v, axis=0, keepdims=True)
    e = jnp.exp(v - m)
    return e / jnp.sum(e, axis=0, keepdims=True)


def _k1_body(x_ref, pe_ref, lat_ref, rw_ref, sw_ref,
             slg_ref, slb_ref, qg_ref, qb_ref, kvg_ref, kvb_ref,
             wqs_ref, bqs_ref, wks_ref, bks_ref, wvs_ref, bvs_ref,
             p4_ref, m4_ref, p4t_ref,
             sew_ref, seb_ref, oew_ref, oeb_ref, okvg_ref, okvb_ref,
             wko_ref, bko_ref, wvo_ref, bvo_ref,
             sow_ref, sob_ref, fg_ref, fb_ref,
             w1_ref, b1_ref, w2_ref, b2_ref, mg_ref, mb_ref, gw_ref,
             ridx_ref, widx_ref, ww_ref, selw_ref,
             kpo_ref, vpo_ref, ll8_ref, l2_ref, gidx_ref, gww_ref,
             qbs_s, lr0_s, kp4_s, vp4_s, m_s, l_s, acc_s,
             *, n_t, tile):
    S, KR, KW, E, TOPK, HKR = 8, 4, 2, 16, 2, 64
    t = pl.program_id(1)

    @pl.when(t == 0)
    def _s0():
        lat = lat_ref[0]
        rs_col = _dx(lat, rw_ref[...])
        sel_r, rlog, ridx = _topk_from_col(rs_col, S, KR)
        ridx_ref[0] = ridx * jnp.ones((KR, S), F32)
        lr0 = _dx(sel_r, lat) * rlog
        lr0_s[...] = lr0
        qp = _dot(_ln(lr0, qg_ref[...], qb_ref[...]), wqs_ref[...]) \
            + bqs_ref[...]
        qbs_s[...] = _dx(p4_ref[...], qp) * m4_ref[...]
        kvln = _ln(lr0, kvg_ref[...], kvb_ref[...])
        kp4_s[...] = _dot(kvln, wks_ref[...]) + bks_ref[...]
        vp4_s[...] = _dot(kvln, wvs_ref[...]) + bvs_ref[...]
        lns = _ln(lat, slg_ref[...], slb_ref[...])
        ss_col = _dx(lns, sw_ref[...])
        sel_w, wlog, widx = _topk_from_col(ss_col, S, KW)
        widx_ref[0] = widx * jnp.ones((KW, S), F32)
        selw_ref[0] = sel_w
        ww_ref[0] = _softmax_col(wlog) * jnp.ones((KW, S), F32)
        m_s[...] = jnp.full((HKR, 128), -jnp.inf, F32)
        l_s[...] = jnp.zeros((HKR, 128), F32)
        acc_s[...] = jnp.zeros((HKR, sew_ref.shape[1]), F32)

    xt = x_ref[0]
    pe = pe_ref[...]
    lsx = _dot(xt, sew_ref[...]) + seb_ref[...] + pe
    ln_s = _ln(lsx, kvg_ref[...], kvb_ref[...])
    kpt = _dot(ln_s, wks_ref[...]) + bks_ref[...]
    vpt = _dot(ln_s, wvs_ref[...]) + bvs_ref[...]
    lout = _dot(xt, oew_ref[...]) + oeb_ref[...] + pe
    ln_o = _ln(lout, okvg_ref[...], okvb_ref[...])
    kpo_ref[0] = (_dot(ln_o, wko_ref[...]) + bko_ref[...]).astype(BF16)
    vpo_ref[0] = (_dot(ln_o, wvo_ref[...]) + bvo_ref[...]).astype(BF16)

    s = jax.lax.dot_general(qbs_s[...].astype(BF16), kpt.astype(BF16),
                            DIMS_NT, preferred_element_type=F32) * 0.125
    m_old = m_s[:, :1]
    m_new = jnp.maximum(m_old, jnp.max(s, axis=1, keepdims=True))
    p = jnp.exp(s - m_new)
    scale = jnp.exp(m_old - m_new)
    l_s[...] = (l_s[:, :1] * scale + jnp.sum(p, axis=1, keepdims=True)) \
        + jnp.zeros((HKR, 128), F32)
    acc_s[...] = acc_s[...] * scale + _dot(p, vpt)
    m_s[...] = m_new + jnp.zeros((HKR, 128), F32)

    @pl.when(t == n_t - 1)
    def _fin():
        ll8_ref[0] = lout[tile - 8:, :]
        s4 = jax.lax.dot_general(qbs_s[...], kp4_s[...], DIMS_NT,
                                 preferred_element_type=F32) * 0.125
        m_f = jnp.maximum(m_s[:, :1], jnp.max(s4, axis=1, keepdims=True))
        e4 = jnp.exp(s4 - m_f)
        sc = jnp.exp(m_s[:, :1] - m_f)
        l_f = l_s[:, :1] * sc + jnp.sum(e4, axis=1, keepdims=True)
        attn = (acc_s[...] * sc + e4 @ vp4_s[...]) / l_f
        o = _dx(p4t_ref[...], m4_ref[...] * attn)
        l1 = lr0_s[...] + _dot(o, sow_ref[...]) + sob_ref[...]
        hn = _ln(l1, fg_ref[...], fb_ref[...])
        g = _gelu(_dot(hn, w1_ref[...]) + b1_ref[...])
        l2 = l1 + _dot(g, w2_ref[...]) + b2_ref[...]
        l2_ref[0] = l2
        pooled = _ln(jnp.mean(l2, axis=0, keepdims=True),
                     mg_ref[...], mb_ref[...])
        glog_col = _to_col(_dx(pooled, gw_ref[...]), E)
        _, gval, gidx = _topk_from_col(glog_col, E, TOPK)
        gidx_ref[0] = gidx * jnp.ones((TOPK, E), F32)
        gww_ref[0] = _softmax_col(gval) * jnp.ones((TOPK, E), F32)


def _k2_body(gidx_sref, wsm_sref, l2_ref, expw_ref, expb_ref,
             selw_ref, ww_ref, kpo_ref, vpo_ref, ll8_ref,
             okvg_ref, okvb_ref, wko_ref, bko_ref, wvo_ref, bvo_ref,
             oqg_ref, oqb_ref, wqo_ref, bqo_ref, m1_ref,
             oow_ref, oob_ref, ofg_ref, ofb_ref,
             w1_ref, b1_ref, w2_ref, b2_ref, pw_ref, pb_ref,
             state_ref, y_ref, acc_s):
    S, KW, HH, TOPK = 8, 2, 16, 2
    b = pl.program_id(0)
    k = pl.program_id(1)

    @pl.when(k == 0)
    def _():
        acc_s[...] = l2_ref[0]

    wk = wsm_sref[b * TOPK + k]
    yk = jax.nn.relu(_dot(l2_ref[0], expw_ref[0]) + expb_ref[0])
    acc_s[...] += wk * yk

    @pl.when(k == TOPK - 1)
    def _fin():
        l3 = acc_s[...]
        mean_row = jnp.mean(l3, axis=0, keepdims=True)
        ww_col = jnp.mean(ww_ref[0], axis=1, keepdims=True)
        ww_row = _to_row(ww_col, KW)
        c_row = _dx(ww_row, selw_ref[0])
        c_col = _to_col(c_row, S)
        state = _dx(c_col, mean_row)
        state_ref[0] = state
        lnst = _ln(state, okvg_ref[...], okvb_ref[...])
        kpst = _dot(lnst, wko_ref[...]) + bko_ref[...]
        vpst = _dot(lnst, wvo_ref[...]) + bvo_ref[...]
        ll = ll8_ref[0][7:8, :]
        q2 = _dot(_ln(ll, oqg_ref[...], oqb_ref[...]), wqo_ref[...]) \
            + bqo_ref[...]
        qbo = _dx(jnp.ones((HH, 1), F32), q2) * m1_ref[...]
        s_main = jax.lax.dot_general(qbo.astype(BF16), kpo_ref[0],
                                     DIMS_NT,
                                     preferred_element_type=F32) * 0.125
        s_st = jax.lax.dot_general(qbo, kpst, DIMS_NT,
                                   preferred_element_type=F32) * 0.125
        m = jnp.maximum(jnp.max(s_main, axis=1, keepdims=True),
                        jnp.max(s_st, axis=1, keepdims=True))
        e_main = jnp.exp(s_main - m)
        e_st = jnp.exp(s_st - m)
        l = jnp.sum(e_main, axis=1, keepdims=True) \
            + jnp.sum(e_st, axis=1, keepdims=True)
        attn = (jnp.dot(e_main.astype(BF16), vpo_ref[0],
                        preferred_element_type=F32)
                + e_st @ vpst) / l
        o = _dx(jnp.ones((1, HH), F32), m1_ref[...] * attn)
        l1 = ll + _dot(o, oow_ref[...]) + oob_ref[...]
        hn = _ln(l1, ofg_ref[...], ofb_ref[...])
        g = _gelu(_dot(hn, w1_ref[...]) + b1_ref[...])
        lf = l1 + _dot(g, w2_ref[...]) + b2_ref[...]
        y_ref[0] = _dot(lf, pw_ref[...]) + pb_ref[...]


def _full_spec(shape):
    return pl.BlockSpec(shape, lambda *a: tuple(0 for _ in shape))


def kernel(x, state_flat, params):
    p = params
    B, T, DIN = x.shape
    D = p['se_w'].shape[0]
    S = state_flat.shape[1] // D
    E = p['gate_w'].shape[0]
    H, TOPK, KR, KW = 16, 2, 4, 2
    TILE = 256
    n_t = T // TILE

    f32 = jnp.float32
    latent = state_flat.reshape(B, S, D)

    pos = jnp.arange(T, dtype=f32)[:, None]
    f = float(S) ** (jnp.arange(D // 2).astype(f32) / (D // 2))
    pe = jnp.concatenate([jnp.sin(pos / f), jnp.cos(pos / f)], axis=-1)

    def row(v):
        return v.reshape(1, -1).astype(f32)

    sew_t = p['se_w'].T
    oew_t = p['oe_w'].T
    wq_s_t = p['smha_in_w'][:D].T
    wk_s_t = p['smha_in_w'][D:2 * D].T
    wv_s_t = p['smha_in_w'][2 * D:].T
    bq_s, bk_s, bv_s = (row(p['smha_in_b'][i * D:(i + 1) * D]) for i in range(3))
    wq_o_t = p['omha_in_w'][:D].T
    wk_o_t = p['omha_in_w'][D:2 * D].T
    wv_o_t = p['omha_in_w'][2 * D:].T
    bq_o, bk_o, bv_o = (row(p['omha_in_b'][i * D:(i + 1) * D]) for i in range(3))
    exp_b3 = p['exp_b'][:, None, :]

    DH = D // H
    r64 = jnp.arange(H * KR)
    c = jnp.arange(D)
    P4 = (r64[:, None] % KR == jnp.arange(KR)[None, :]).astype(f32)
    M4 = ((c[None, :] // DH) == (r64[:, None] // KR)).astype(f32)
    P4T = P4.T
    M1 = ((c[None, :] // DH) == jnp.arange(H)[:, None]).astype(f32)

    def bspec(rows, cols):
        return pl.BlockSpec((1, rows, cols), lambda b, *a: (b, 0, 0))

    k1_out = pl.pallas_call(
        functools.partial(_k1_body, n_t=n_t, tile=TILE),
        grid=(B, n_t),
        in_specs=[pl.BlockSpec((1, TILE, DIN), lambda b, t: (b, t, 0)),
                  pl.BlockSpec((TILE, D), lambda b, t: (t, 0)),
                  bspec(S, D), _full_spec((D, 1)), _full_spec((D, 1)),
                  _full_spec((1, D)), _full_spec((1, D)),
                  _full_spec((1, D)), _full_spec((1, D)),
                  _full_spec((1, D)), _full_spec((1, D)),
                  _full_spec((D, D)), _full_spec((1, D)),
                  _full_spec((D, D)), _full_spec((1, D)),
                  _full_spec((D, D)), _full_spec((1, D)),
                  _full_spec((H * KR, KR)), _full_spec((H * KR, D)),
                  _full_spec((KR, H * KR)),
                  _full_spec((DIN, D)), _full_spec((1, D)),
                  _full_spec((DIN, D)), _full_spec((1, D)),
                  _full_spec((1, D)), _full_spec((1, D)),
                  _full_spec((D, D)), _full_spec((1, D)),
                  _full_spec((D, D)), _full_spec((1, D)),
                  _full_spec((D, D)), _full_spec((1, D)),
                  _full_spec((1, D)), _full_spec((1, D)),
                  _full_spec((D, D)), _full_spec((1, D)),
                  _full_spec((D, D)), _full_spec((1, D)),
                  _full_spec((1, D)), _full_spec((1, D)),
                  _full_spec((D, E))],
        out_specs=[bspec(KR, S), bspec(KW, S), bspec(KW, S), bspec(KW, S),
                   pl.BlockSpec((1, TILE, D), lambda b, t: (b, t, 0)),
                   pl.BlockSpec((1, TILE, D), lambda b, t: (b, t, 0)),
                   bspec(8, D), bspec(KR, D), bspec(TOPK, E), bspec(TOPK, E)],
        out_shape=[jax.ShapeDtypeStruct((B, KR, S), f32),
                   jax.ShapeDtypeStruct((B, KW, S), f32),
                   jax.ShapeDtypeStruct((B, KW, S), f32),
                   jax.ShapeDtypeStruct((B, KW, S), f32),
                   jax.ShapeDtypeStruct((B, T, D), jnp.bfloat16),
                   jax.ShapeDtypeStruct((B, T, D), jnp.bfloat16),
                   jax.ShapeDtypeStruct((B, 8, D), f32),
                   jax.ShapeDtypeStruct((B, KR, D), f32),
                   jax.ShapeDtypeStruct((B, TOPK, E), f32),
                   jax.ShapeDtypeStruct((B, TOPK, E), f32)],
        scratch_shapes=[pltpu.VMEM((H * KR, D), f32),
                        pltpu.VMEM((KR, D), f32),
                        pltpu.VMEM((KR, D), f32),
                        pltpu.VMEM((KR, D), f32),
                        pltpu.VMEM((H * KR, 128), f32),
                        pltpu.VMEM((H * KR, 128), f32),
                        pltpu.VMEM((H * KR, D), f32)],
        interpret=INTERP,
    )(x, pe, latent, p['read_w'].T, p['slot_w'].T,
      row(p['sln_slot_g']), row(p['sln_slot_b']),
      row(p['sln_q_g']), row(p['sln_q_b']),
      row(p['sln_kv_g']), row(p['sln_kv_b']),
      wq_s_t, bq_s, wk_s_t, bk_s, wv_s_t, bv_s,
      P4, M4, P4T,
      sew_t, row(p['se_b']), oew_t, row(p['oe_b']),
      row(p['oln_kv_g']), row(p['oln_kv_b']),
      wk_o_t, bk_o, wv_o_t, bv_o,
      p['smha_out_w'].T, row(p['smha_out_b']),
      row(p['sln_ffn_g']), row(p['sln_ffn_b']),
      p['sffn_w1'].T, row(p['sffn_b1']), p['sffn_w2'].T, row(p['sffn_b2']),
      row(p['sln_moe_g']), row(p['sln_moe_b']), p['gate_w'].T)

    (ridx_b, widx_b, ww_b, selw, kpo, vpo, ll8, l2,
     gidx_b, gw_b) = k1_out

    gidx = gidx_b[:, :, 0].astype(jnp.int32)
    gidx_flat = gidx.reshape(-1)
    w_flat = gw_b[:, :, 0].reshape(-1)

    grid_spec = pltpu.PrefetchScalarGridSpec(
        num_scalar_prefetch=2,
        grid=(B, TOPK),
        in_specs=[pl.BlockSpec((1, KR, D), lambda b, k, gref, wref: (b, 0, 0)),
                  pl.BlockSpec((1, D, D),
                               lambda b, k, gref, wref: (gref[b * 2 + k], 0, 0)),
                  pl.BlockSpec((1, 1, D),
                               lambda b, k, gref, wref: (gref[b * 2 + k], 0, 0)),
                  pl.BlockSpec((1, KW, S), lambda b, k, gref, wref: (b, 0, 0)),
                  pl.BlockSpec((1, KW, S), lambda b, k, gref, wref: (b, 0, 0)),
                  pl.BlockSpec((1, T, D), lambda b, k, gref, wref: (b, 0, 0)),
                  pl.BlockSpec((1, T, D), lambda b, k, gref, wref: (b, 0, 0)),
                  pl.BlockSpec((1, 8, D), lambda b, k, gref, wref: (b, 0, 0))]
        + [pl.BlockSpec(s, lambda b, k, gref, wref, _s=s:
                        tuple(0 for _ in _s))
           for s in [(1, D), (1, D), (D, D), (1, D), (D, D), (1, D),
                     (1, D), (1, D), (D, D), (1, D), (H, D),
                     (D, D), (1, D), (1, D), (1, D),
                     (D, D), (1, D), (D, D), (1, D),
                     (D, p['outp_w'].shape[0]), (1, p['outp_w'].shape[0])]],
        out_specs=[pl.BlockSpec((1, S, D), lambda b, k, gref, wref: (b, 0, 0)),
                   pl.BlockSpec((1, 1, p['outp_w'].shape[0]),
                                lambda b, k, gref, wref: (b, 0, 0))],
        scratch_shapes=[pltpu.VMEM((KR, D), f32)],
    )
    state3, y3 = pl.pallas_call(
        _k2_body,
        grid_spec=grid_spec,
        out_shape=[jax.ShapeDtypeStruct((B, S, D), f32),
                   jax.ShapeDtypeStruct((B, 1, p['outp_w'].shape[0]), f32)],
        interpret=INTERP,
    )(gidx_flat, w_flat, l2, p['exp_w'], exp_b3,
      selw, ww_b, kpo, vpo, ll8,
      row(p['oln_kv_g']), row(p['oln_kv_b']),
      wk_o_t, bk_o, wv_o_t, bv_o,
      row(p['oln_q_g']), row(p['oln_q_b']), wq_o_t, bq_o, M1,
      p['omha_out_w'].T, row(p['omha_out_b']),
      row(p['oln_ffn_g']), row(p['oln_ffn_b']),
      p['offn_w1'].T, row(p['offn_b1']), p['offn_w2'].T, row(p['offn_b2']),
      p['outp_w'].T, row(p['outp_b']))

    y = y3[:, 0, :]
    read_idx = ridx_b[:, :, 0].astype(jnp.int32)
    write_idx = widx_b[:, :, 0].astype(jnp.int32)
    state_out = state3.reshape(B, S * D)
    return y, gidx, read_idx, write_idx, state_out


# speedup vs baseline: 1.2121x; 1.2121x over previous
import functools

import jax
import jax.numpy as jnp
from jax.experimental import pallas as pl
from jax.experimental.pallas import tpu as pltpu

INTERP = False

F32 = jnp.float32
BF16 = jnp.bfloat16
DIMS_NT = (((1,), (1,)), ((), ()))


def _dx(a, b):
    return jnp.dot(a, b, precision=jax.lax.Precision.HIGHEST)


def _dot(a, b):
    return jnp.dot(a.astype(BF16), b.astype(BF16), preferred_element_type=F32)


def _dotb(a, b_ref):
    return jnp.dot(a.astype(BF16), b_ref[...], preferred_element_type=F32)


def _ln(x, g, b, eps=1e-5):
    m = jnp.mean(x, axis=-1, keepdims=True)
    v = jnp.mean((x - m) ** 2, axis=-1, keepdims=True)
    return (x - m) * jax.lax.rsqrt(v + eps) * g + b


def _eye(n):
    ii = jax.lax.broadcasted_iota(jnp.int32, (n, n), 0)
    jj = jax.lax.broadcasted_iota(jnp.int32, (n, n), 1)
    return (ii == jj).astype(F32)


def _to_row(col, n):
    bm = col * jnp.ones((n, n), F32)
    return _dx(jnp.ones((1, n), F32), _eye(n) * bm)


def _to_col(row, n):
    am = _dx(jnp.ones((n, 1), F32), row)
    return _dx(_eye(n) * am, jnp.ones((n, 1), F32))


def _topk_from_col(s_col, n, k):
    ones_nn = jnp.ones((n, n), F32)
    bm = s_col * ones_nn
    am = _dx(ones_nn, _eye(n) * bm)
    ii = jax.lax.broadcasted_iota(jnp.int32, (n, n), 0)
    jj = jax.lax.broadcasted_iota(jnp.int32, (n, n), 1)
    gt = (am > bm).astype(F32)
    tie = ((am == bm) & (jj < ii)).astype(F32)
    rank_col = jnp.sum(gt + tie, axis=1, keepdims=True)
    rank_row = _to_row(rank_col, n)
    rr = rank_row + jnp.zeros((k, n), F32)
    kio = jax.lax.broadcasted_iota(jnp.int32, (k, n), 0).astype(F32)
    sel = (rr == kio).astype(F32)
    j_col = jax.lax.broadcasted_iota(jnp.int32, (n, 1), 0).astype(F32)
    idx_col = _dx(sel, j_col)
    vals_col = _dx(sel, s_col)
    return sel, vals_col, idx_col


def _gelu(x):
    return 0.5 * x * (1.0 + jax.lax.erf(x * 0.7071067811865476))


def _softmax_col(v):
    m = jnp.max(v, axis=0, keepdims=True)
    e = jnp.exp(v - m)
    return e / jnp.sum(e, axis=0, keepdims=True)


def _k1_body(x_ref, pe_ref, lat_ref, rw_ref, sw_ref,
             slg_ref, slb_ref, qg_ref, qb_ref, kvg_ref, kvb_ref,
             wqs_ref, bqs_ref, wks_ref, bks_ref, wvs_ref, bvs_ref,
             p4_ref, m4_ref, p4t_ref,
             sew_ref, seb_ref, oew_ref, oeb_ref, okvg_ref, okvb_ref,
             wko_ref, bko_ref, wvo_ref, bvo_ref,
             sow_ref, sob_ref, fg_ref, fb_ref,
             w1_ref, b1_ref, w2_ref, b2_ref, mg_ref, mb_ref, gw_ref,
             ridx_ref, widx_ref, ww_ref, selw_ref,
             kpo_ref, vpo_ref, ll8_ref, l2_ref, gidx_ref, gww_ref,
             qbs_s, lr0_s, kp4_s, vp4_s, m_s, l_s, acc_s,
             *, n_t, tile):
    S, KR, KW, E, TOPK, HKR = 8, 4, 2, 16, 2, 64
    t = pl.program_id(1)

    @pl.when(t == 0)
    def _s0():
        lat = lat_ref[0]
        rs_col = _dx(lat, rw_ref[...])
        sel_r, rlog, ridx = _topk_from_col(rs_col, S, KR)
        ridx_ref[0] = ridx * jnp.ones((KR, S), F32)
        lr0 = _dx(sel_r, lat) * rlog
        lr0_s[...] = lr0
        qp = _dotb(_ln(lr0, qg_ref[...], qb_ref[...]), wqs_ref) \
            + bqs_ref[...]
        qbs_s[...] = _dx(p4_ref[...], qp) * m4_ref[...]
        kvln = _ln(lr0, kvg_ref[...], kvb_ref[...])
        kp4_s[...] = _dotb(kvln, wks_ref) + bks_ref[...]
        vp4_s[...] = _dotb(kvln, wvs_ref) + bvs_ref[...]
        lns = _ln(lat, slg_ref[...], slb_ref[...])
        ss_col = _dx(lns, sw_ref[...])
        sel_w, wlog, widx = _topk_from_col(ss_col, S, KW)
        widx_ref[0] = widx * jnp.ones((KW, S), F32)
        selw_ref[0] = sel_w
        ww_ref[0] = _softmax_col(wlog) * jnp.ones((KW, S), F32)
        m_s[...] = jnp.full((HKR, 128), -jnp.inf, F32)
        l_s[...] = jnp.zeros((HKR, 128), F32)
        acc_s[...] = jnp.zeros((HKR, sew_ref.shape[1]), F32)

    xt = x_ref[0]
    pe = pe_ref[...]
    lsx = jnp.dot(xt, sew_ref[...], preferred_element_type=F32) \
        + seb_ref[...] + pe
    ln_s = _ln(lsx, kvg_ref[...], kvb_ref[...])
    kpt = _dotb(ln_s, wks_ref) + bks_ref[...]
    vpt = _dotb(ln_s, wvs_ref) + bvs_ref[...]
    lout = jnp.dot(xt, oew_ref[...], preferred_element_type=F32) \
        + oeb_ref[...] + pe
    ln_o = _ln(lout, okvg_ref[...], okvb_ref[...])
    kpo_ref[0] = (_dotb(ln_o, wko_ref) + bko_ref[...]).astype(BF16)
    vpo_ref[0] = (_dotb(ln_o, wvo_ref) + bvo_ref[...]).astype(BF16)

    s = jax.lax.dot_general(qbs_s[...].astype(BF16), kpt.astype(BF16),
                            DIMS_NT, preferred_element_type=F32) * 0.125
    m_old = m_s[:, :1]
    m_new = jnp.maximum(m_old, jnp.max(s, axis=1, keepdims=True))
    p = jnp.exp(s - m_new)
    scale = jnp.exp(m_old - m_new)
    l_s[...] = (l_s[:, :1] * scale + jnp.sum(p, axis=1, keepdims=True)) \
        + jnp.zeros((HKR, 128), F32)
    acc_s[...] = acc_s[...] * scale + _dot(p, vpt)
    m_s[...] = m_new + jnp.zeros((HKR, 128), F32)

    @pl.when(t == n_t - 1)
    def _fin():
        ll8_ref[0] = lout[tile - 8:, :]
        s4 = jax.lax.dot_general(qbs_s[...], kp4_s[...], DIMS_NT,
                                 preferred_element_type=F32) * 0.125
        m_f = jnp.maximum(m_s[:, :1], jnp.max(s4, axis=1, keepdims=True))
        e4 = jnp.exp(s4 - m_f)
        sc = jnp.exp(m_s[:, :1] - m_f)
        l_f = l_s[:, :1] * sc + jnp.sum(e4, axis=1, keepdims=True)
        attn = (acc_s[...] * sc + e4 @ vp4_s[...]) / l_f
        o = _dx(p4t_ref[...], m4_ref[...] * attn)
        l1 = lr0_s[...] + _dotb(o, sow_ref) + sob_ref[...]
        hn = _ln(l1, fg_ref[...], fb_ref[...])
        g = _gelu(_dotb(hn, w1_ref) + b1_ref[...])
        l2 = l1 + _dotb(g, w2_ref) + b2_ref[...]
        l2_ref[0] = l2
        pooled = _ln(jnp.mean(l2, axis=0, keepdims=True),
                     mg_ref[...], mb_ref[...])
        glog_col = _to_col(_dx(pooled, gw_ref[...]), E)
        _, gval, gidx = _topk_from_col(glog_col, E, TOPK)
        gidx_ref[0] = gidx * jnp.ones((TOPK, E), F32)
        gww_ref[0] = _softmax_col(gval) * jnp.ones((TOPK, E), F32)


def _k2_body(gidx_sref, wsm_sref, l2_ref, expw_ref, expb_ref,
             selw_ref, ww_ref, kpo_ref, vpo_ref, ll8_ref,
             okvg_ref, okvb_ref, wko_ref, bko_ref, wvo_ref, bvo_ref,
             oqg_ref, oqb_ref, wqo_ref, bqo_ref, m1_ref,
             oow_ref, oob_ref, ofg_ref, ofb_ref,
             w1_ref, b1_ref, w2_ref, b2_ref, pw_ref, pb_ref,
             state_ref, y_ref, acc_s):
    S, KW, HH, TOPK = 8, 2, 16, 2
    b = pl.program_id(0)
    k = pl.program_id(1)

    @pl.when(k == 0)
    def _():
        acc_s[...] = l2_ref[0]

    wk = wsm_sref[b * TOPK + k]
    yk = jax.nn.relu(_dot(l2_ref[0], expw_ref[0]) + expb_ref[0])
    acc_s[...] += wk * yk

    @pl.when(k == TOPK - 1)
    def _fin():
        l3 = acc_s[...]
        mean_row = jnp.mean(l3, axis=0, keepdims=True)
        ww_col = jnp.mean(ww_ref[0], axis=1, keepdims=True)
        ww_row = _to_row(ww_col, KW)
        c_row = _dx(ww_row, selw_ref[0])
        c_col = _to_col(c_row, S)
        state = _dx(c_col, mean_row)
        state_ref[0] = state
        lnst = _ln(state, okvg_ref[...], okvb_ref[...])
        kpst = _dotb(lnst, wko_ref) + bko_ref[...]
        vpst = _dotb(lnst, wvo_ref) + bvo_ref[...]
        ll = ll8_ref[0][7:8, :]
        q2 = _dotb(_ln(ll, oqg_ref[...], oqb_ref[...]), wqo_ref) \
            + bqo_ref[...]
        qbo = _dx(jnp.ones((HH, 1), F32), q2) * m1_ref[...]
        s_main = jax.lax.dot_general(qbo.astype(BF16), kpo_ref[0],
                                     DIMS_NT,
                                     preferred_element_type=F32) * 0.125
        s_st = jax.lax.dot_general(qbo, kpst, DIMS_NT,
                                   preferred_element_type=F32) * 0.125
        m = jnp.maximum(jnp.max(s_main, axis=1, keepdims=True),
                        jnp.max(s_st, axis=1, keepdims=True))
        e_main = jnp.exp(s_main - m)
        e_st = jnp.exp(s_st - m)
        l = jnp.sum(e_main, axis=1, keepdims=True) \
            + jnp.sum(e_st, axis=1, keepdims=True)
        attn = (jnp.dot(e_main.astype(BF16), vpo_ref[0],
                        preferred_element_type=F32)
                + e_st @ vpst) / l
        o = _dx(jnp.ones((1, HH), F32), m1_ref[...] * attn)
        l1 = ll + _dotb(o, oow_ref) + oob_ref[...]
        hn = _ln(l1, ofg_ref[...], ofb_ref[...])
        g = _gelu(_dotb(hn, w1_ref) + b1_ref[...])
        lf = l1 + _dotb(g, w2_ref) + b2_ref[...]
        y_ref[0] = _dotb(lf, pw_ref) + pb_ref[...]


def _full_spec(shape):
    return pl.BlockSpec(shape, lambda *a: tuple(0 for _ in shape))


def kernel(x, state_flat, params):
    p = params
    B, T, DIN = x.shape
    D = p['se_w'].shape[0]
    S = state_flat.shape[1] // D
    E = p['gate_w'].shape[0]
    H, TOPK, KR, KW = 16, 2, 4, 2
    TILE = 512
    n_t = T // TILE

    f32 = jnp.float32
    latent = state_flat.reshape(B, S, D)

    pos = jnp.arange(T, dtype=f32)[:, None]
    f = float(S) ** (jnp.arange(D // 2).astype(f32) / (D // 2))
    pe = jnp.concatenate([jnp.sin(pos / f), jnp.cos(pos / f)], axis=-1)

    def row(v):
        return v.reshape(1, -1).astype(f32)

    sew_t = p['se_w'].T
    oew_t = p['oe_w'].T
    wq_s_t = p['smha_in_w'][:D].T
    wk_s_t = p['smha_in_w'][D:2 * D].T
    wv_s_t = p['smha_in_w'][2 * D:].T
    bq_s, bk_s, bv_s = (row(p['smha_in_b'][i * D:(i + 1) * D]) for i in range(3))
    wq_o_t = p['omha_in_w'][:D].T
    wk_o_t = p['omha_in_w'][D:2 * D].T
    wv_o_t = p['omha_in_w'][2 * D:].T
    bq_o, bk_o, bv_o = (row(p['omha_in_b'][i * D:(i + 1) * D]) for i in range(3))
    exp_b3 = p['exp_b'][:, None, :]
    bf16 = jnp.bfloat16
    wk_s_b = wk_s_t.astype(bf16)
    wv_s_b = wv_s_t.astype(bf16)
    wk_o_b = wk_o_t.astype(bf16)
    wv_o_b = wv_o_t.astype(bf16)

    DH = D // H
    r64 = jnp.arange(H * KR)
    c = jnp.arange(D)
    P4 = (r64[:, None] % KR == jnp.arange(KR)[None, :]).astype(f32)
    M4 = ((c[None, :] // DH) == (r64[:, None] // KR)).astype(f32)
    P4T = P4.T
    M1 = ((c[None, :] // DH) == jnp.arange(H)[:, None]).astype(f32)

    def bspec(rows, cols):
        return pl.BlockSpec((1, rows, cols), lambda b, *a: (b, 0, 0))

    k1_out = pl.pallas_call(
        functools.partial(_k1_body, n_t=n_t, tile=TILE),
        grid=(B, n_t),
        in_specs=[pl.BlockSpec((1, TILE, DIN), lambda b, t: (b, t, 0)),
                  pl.BlockSpec((TILE, D), lambda b, t: (t, 0)),
                  bspec(S, D), _full_spec((D, 1)), _full_spec((D, 1)),
                  _full_spec((1, D)), _full_spec((1, D)),
                  _full_spec((1, D)), _full_spec((1, D)),
                  _full_spec((1, D)), _full_spec((1, D)),
                  _full_spec((D, D)), _full_spec((1, D)),
                  _full_spec((D, D)), _full_spec((1, D)),
                  _full_spec((D, D)), _full_spec((1, D)),
                  _full_spec((H * KR, KR)), _full_spec((H * KR, D)),
                  _full_spec((KR, H * KR)),
                  _full_spec((DIN, D)), _full_spec((1, D)),
                  _full_spec((DIN, D)), _full_spec((1, D)),
                  _full_spec((1, D)), _full_spec((1, D)),
                  _full_spec((D, D)), _full_spec((1, D)),
                  _full_spec((D, D)), _full_spec((1, D)),
                  _full_spec((D, D)), _full_spec((1, D)),
                  _full_spec((1, D)), _full_spec((1, D)),
                  _full_spec((D, D)), _full_spec((1, D)),
                  _full_spec((D, D)), _full_spec((1, D)),
                  _full_spec((1, D)), _full_spec((1, D)),
                  _full_spec((D, E))],
        out_specs=[bspec(KR, S), bspec(KW, S), bspec(KW, S), bspec(KW, S),
                   pl.BlockSpec((1, TILE, D), lambda b, t: (b, t, 0)),
                   pl.BlockSpec((1, TILE, D), lambda b, t: (b, t, 0)),
                   bspec(8, D), bspec(KR, D), bspec(TOPK, E), bspec(TOPK, E)],
        out_shape=[jax.ShapeDtypeStruct((B, KR, S), f32),
                   jax.ShapeDtypeStruct((B, KW, S), f32),
                   jax.ShapeDtypeStruct((B, KW, S), f32),
                   jax.ShapeDtypeStruct((B, KW, S), f32),
                   jax.ShapeDtypeStruct((B, T, D), jnp.bfloat16),
                   jax.ShapeDtypeStruct((B, T, D), jnp.bfloat16),
                   jax.ShapeDtypeStruct((B, 8, D), f32),
                   jax.ShapeDtypeStruct((B, KR, D), f32),
                   jax.ShapeDtypeStruct((B, TOPK, E), f32),
                   jax.ShapeDtypeStruct((B, TOPK, E), f32)],
        scratch_shapes=[pltpu.VMEM((H * KR, D), f32),
                        pltpu.VMEM((KR, D), f32),
                        pltpu.VMEM((KR, D), f32),
                        pltpu.VMEM((KR, D), f32),
                        pltpu.VMEM((H * KR, 128), f32),
                        pltpu.VMEM((H * KR, 128), f32),
                        pltpu.VMEM((H * KR, D), f32)],
        interpret=INTERP,
    )(x.astype(bf16), pe, latent, p['read_w'].T, p['slot_w'].T,
      row(p['sln_slot_g']), row(p['sln_slot_b']),
      row(p['sln_q_g']), row(p['sln_q_b']),
      row(p['sln_kv_g']), row(p['sln_kv_b']),
      wq_s_t.astype(bf16), bq_s, wk_s_b, bk_s, wv_s_b, bv_s,
      P4, M4, P4T,
      sew_t.astype(bf16), row(p['se_b']), oew_t.astype(bf16), row(p['oe_b']),
      row(p['oln_kv_g']), row(p['oln_kv_b']),
      wk_o_b, bk_o, wv_o_b, bv_o,
      p['smha_out_w'].T.astype(bf16), row(p['smha_out_b']),
      row(p['sln_ffn_g']), row(p['sln_ffn_b']),
      p['sffn_w1'].T.astype(bf16), row(p['sffn_b1']),
      p['sffn_w2'].T.astype(bf16), row(p['sffn_b2']),
      row(p['sln_moe_g']), row(p['sln_moe_b']), p['gate_w'].T)

    (ridx_b, widx_b, ww_b, selw, kpo, vpo, ll8, l2,
     gidx_b, gw_b) = k1_out

    gidx = gidx_b[:, :, 0].astype(jnp.int32)
    gidx_flat = gidx.reshape(-1)
    w_flat = gw_b[:, :, 0].reshape(-1)

    grid_spec = pltpu.PrefetchScalarGridSpec(
        num_scalar_prefetch=2,
        grid=(B, TOPK),
        in_specs=[pl.BlockSpec((1, KR, D), lambda b, k, gref, wref: (b, 0, 0)),
                  pl.BlockSpec((1, D, D),
                               lambda b, k, gref, wref: (gref[b * 2 + k], 0, 0)),
                  pl.BlockSpec((1, 1, D),
                               lambda b, k, gref, wref: (gref[b * 2 + k], 0, 0)),
                  pl.BlockSpec((1, KW, S), lambda b, k, gref, wref: (b, 0, 0)),
                  pl.BlockSpec((1, KW, S), lambda b, k, gref, wref: (b, 0, 0)),
                  pl.BlockSpec((1, T, D), lambda b, k, gref, wref: (b, 0, 0)),
                  pl.BlockSpec((1, T, D), lambda b, k, gref, wref: (b, 0, 0)),
                  pl.BlockSpec((1, 8, D), lambda b, k, gref, wref: (b, 0, 0))]
        + [pl.BlockSpec(s, lambda b, k, gref, wref, _s=s:
                        tuple(0 for _ in _s))
           for s in [(1, D), (1, D), (D, D), (1, D), (D, D), (1, D),
                     (1, D), (1, D), (D, D), (1, D), (H, D),
                     (D, D), (1, D), (1, D), (1, D),
                     (D, D), (1, D), (D, D), (1, D),
                     (D, p['outp_w'].shape[0]), (1, p['outp_w'].shape[0])]],
        out_specs=[pl.BlockSpec((1, S, D), lambda b, k, gref, wref: (b, 0, 0)),
                   pl.BlockSpec((1, 1, p['outp_w'].shape[0]),
                                lambda b, k, gref, wref: (b, 0, 0))],
        scratch_shapes=[pltpu.VMEM((KR, D), f32)],
    )
    state3, y3 = pl.pallas_call(
        _k2_body,
        grid_spec=grid_spec,
        out_shape=[jax.ShapeDtypeStruct((B, S, D), f32),
                   jax.ShapeDtypeStruct((B, 1, p['outp_w'].shape[0]), f32)],
        interpret=INTERP,
    )(gidx_flat, w_flat, l2, p['exp_w'], exp_b3,
      selw, ww_b, kpo, vpo, ll8,
      row(p['oln_kv_g']), row(p['oln_kv_b']),
      wk_o_b, bk_o, wv_o_b, bv_o,
      row(p['oln_q_g']), row(p['oln_q_b']), wq_o_t.astype(bf16), bq_o, M1,
      p['omha_out_w'].T.astype(bf16), row(p['omha_out_b']),
      row(p['oln_ffn_g']), row(p['oln_ffn_b']),
      p['offn_w1'].T.astype(bf16), row(p['offn_b1']),
      p['offn_w2'].T.astype(bf16), row(p['offn_b2']),
      p['outp_w'].T.astype(bf16), row(p['outp_b']))

    y = y3[:, 0, :]
    read_idx = ridx_b[:, :, 0].astype(jnp.int32)
    write_idx = widx_b[:, :, 0].astype(jnp.int32)
    state_out = state3.reshape(B, S * D)
    return y, gidx, read_idx, write_idx, state_out


# speedup vs baseline: 1.2288x; 1.0138x over previous
import functools

import jax
import jax.numpy as jnp
from jax.experimental import pallas as pl
from jax.experimental.pallas import tpu as pltpu

INTERP = False

F32 = jnp.float32
BF16 = jnp.bfloat16
DIMS_NT = (((1,), (1,)), ((), ()))


def _dx(a, b):
    return jnp.dot(a, b, precision=jax.lax.Precision.HIGHEST)


def _dot(a, b):
    return jnp.dot(a.astype(BF16), b.astype(BF16), preferred_element_type=F32)


def _dotb(a, b_ref):
    return jnp.dot(a.astype(BF16), b_ref[...], preferred_element_type=F32)


def _ln(x, g, b, eps=1e-5):
    m = jnp.mean(x, axis=-1, keepdims=True)
    v = jnp.mean((x - m) ** 2, axis=-1, keepdims=True)
    return (x - m) * jax.lax.rsqrt(v + eps) * g + b


def _eye(n):
    ii = jax.lax.broadcasted_iota(jnp.int32, (n, n), 0)
    jj = jax.lax.broadcasted_iota(jnp.int32, (n, n), 1)
    return (ii == jj).astype(F32)


def _to_row(col, n):
    bm = col * jnp.ones((n, n), F32)
    return _dx(jnp.ones((1, n), F32), _eye(n) * bm)


def _to_col(row, n):
    am = _dx(jnp.ones((n, 1), F32), row)
    return _dx(_eye(n) * am, jnp.ones((n, 1), F32))


def _topk_from_col(s_col, n, k):
    ones_nn = jnp.ones((n, n), F32)
    bm = s_col * ones_nn
    am = _dx(ones_nn, _eye(n) * bm)
    ii = jax.lax.broadcasted_iota(jnp.int32, (n, n), 0)
    jj = jax.lax.broadcasted_iota(jnp.int32, (n, n), 1)
    gt = (am > bm).astype(F32)
    tie = ((am == bm) & (jj < ii)).astype(F32)
    rank_col = jnp.sum(gt + tie, axis=1, keepdims=True)
    rank_row = _to_row(rank_col, n)
    rr = rank_row + jnp.zeros((k, n), F32)
    kio = jax.lax.broadcasted_iota(jnp.int32, (k, n), 0).astype(F32)
    sel = (rr == kio).astype(F32)
    j_col = jax.lax.broadcasted_iota(jnp.int32, (n, 1), 0).astype(F32)
    idx_col = _dx(sel, j_col)
    vals_col = _dx(sel, s_col)
    return sel, vals_col, idx_col


def _gelu(x):
    return 0.5 * x * (1.0 + jax.lax.erf(x * 0.7071067811865476))


def _softmax_col(v):
    m = jnp.max(v, axis=0, keepdims=True)
    e = jnp.exp(v - m)
    return e / jnp.sum(e, axis=0, keepdims=True)


def _k1_body(x_ref, invf_ref, lat_ref, rw_ref, sw_ref,
             slg_ref, slb_ref, qg_ref, qb_ref, kvg_ref, kvb_ref,
             wqs_ref, bqs_ref, wks_ref, bks_ref, wvs_ref, bvs_ref,
             p4_ref, m4_ref, p4t_ref,
             sew_ref, seb_ref, oew_ref, oeb_ref, okvg_ref, okvb_ref,
             wko_ref, bko_ref, wvo_ref, bvo_ref,
             sow_ref, sob_ref, fg_ref, fb_ref,
             w1_ref, b1_ref, w2_ref, b2_ref, mg_ref, mb_ref, gw_ref,
             ridx_ref, widx_ref, ww_ref, selw_ref,
             kpo_ref, vpo_ref, ll8_ref, l2_ref, gidx_ref, gww_ref,
             qbs_s, lr0_s, kp4_s, vp4_s, m_s, l_s, acc_s,
             *, n_t, tile):
    S, KR, KW, E, TOPK, HKR = 8, 4, 2, 16, 2, 64
    t = pl.program_id(1)

    @pl.when(t == 0)
    def _s0():
        lat = lat_ref[0]
        rs_col = _dx(lat, rw_ref[...])
        sel_r, rlog, ridx = _topk_from_col(rs_col, S, KR)
        ridx_ref[0] = ridx * jnp.ones((KR, S), F32)
        lr0 = _dx(sel_r, lat) * rlog
        lr0_s[...] = lr0
        qp = _dotb(_ln(lr0, qg_ref[...], qb_ref[...]), wqs_ref) \
            + bqs_ref[...]
        qbs_s[...] = _dx(p4_ref[...], qp) * m4_ref[...]
        kvln = _ln(lr0, kvg_ref[...], kvb_ref[...])
        kp4_s[...] = _dotb(kvln, wks_ref) + bks_ref[...]
        vp4_s[...] = _dotb(kvln, wvs_ref) + bvs_ref[...]
        lns = _ln(lat, slg_ref[...], slb_ref[...])
        ss_col = _dx(lns, sw_ref[...])
        sel_w, wlog, widx = _topk_from_col(ss_col, S, KW)
        widx_ref[0] = widx * jnp.ones((KW, S), F32)
        selw_ref[0] = sel_w
        ww_ref[0] = _softmax_col(wlog) * jnp.ones((KW, S), F32)
        m_s[...] = jnp.full((HKR, 128), -jnp.inf, F32)
        l_s[...] = jnp.zeros((HKR, 128), F32)
        acc_s[...] = jnp.zeros((HKR, sew_ref.shape[1]), F32)

    xt = x_ref[0].astype(BF16)
    pos = (t * tile + jax.lax.broadcasted_iota(jnp.int32, (tile, 1), 0)
           ).astype(F32)
    args = pos * invf_ref[...]
    pe = jnp.concatenate([jnp.sin(args), jnp.cos(args)], axis=1)
    lsx = jnp.dot(xt, sew_ref[...], preferred_element_type=F32) \
        + seb_ref[...] + pe
    ln_s = _ln(lsx, kvg_ref[...], kvb_ref[...])
    kpt = _dotb(ln_s, wks_ref) + bks_ref[...]
    vpt = _dotb(ln_s, wvs_ref) + bvs_ref[...]
    lout = jnp.dot(xt, oew_ref[...], preferred_element_type=F32) \
        + oeb_ref[...] + pe
    ln_o = _ln(lout, okvg_ref[...], okvb_ref[...])
    kpo_ref[0] = (_dotb(ln_o, wko_ref) + bko_ref[...]).astype(BF16)
    vpo_ref[0] = (_dotb(ln_o, wvo_ref) + bvo_ref[...]).astype(BF16)

    s = jax.lax.dot_general(qbs_s[...].astype(BF16), kpt.astype(BF16),
                            DIMS_NT, preferred_element_type=F32) * 0.125
    m_old = m_s[:, :1]
    m_new = jnp.maximum(m_old, jnp.max(s, axis=1, keepdims=True))
    p = jnp.exp(s - m_new)
    scale = jnp.exp(m_old - m_new)
    l_s[...] = (l_s[:, :1] * scale + jnp.sum(p, axis=1, keepdims=True)) \
        + jnp.zeros((HKR, 128), F32)
    acc_s[...] = acc_s[...] * scale + _dot(p, vpt)
    m_s[...] = m_new + jnp.zeros((HKR, 128), F32)

    @pl.when(t == n_t - 1)
    def _fin():
        ll8_ref[0] = lout[tile - 8:, :]
        s4 = jax.lax.dot_general(qbs_s[...], kp4_s[...], DIMS_NT,
                                 preferred_element_type=F32) * 0.125
        m_f = jnp.maximum(m_s[:, :1], jnp.max(s4, axis=1, keepdims=True))
        e4 = jnp.exp(s4 - m_f)
        sc = jnp.exp(m_s[:, :1] - m_f)
        l_f = l_s[:, :1] * sc + jnp.sum(e4, axis=1, keepdims=True)
        attn = (acc_s[...] * sc + e4 @ vp4_s[...]) / l_f
        o = _dx(p4t_ref[...], m4_ref[...] * attn)
        l1 = lr0_s[...] + _dotb(o, sow_ref) + sob_ref[...]
        hn = _ln(l1, fg_ref[...], fb_ref[...])
        g = _gelu(_dotb(hn, w1_ref) + b1_ref[...])
        l2 = l1 + _dotb(g, w2_ref) + b2_ref[...]
        l2_ref[0] = l2
        pooled = _ln(jnp.mean(l2, axis=0, keepdims=True),
                     mg_ref[...], mb_ref[...])
        glog_col = _to_col(_dx(pooled, gw_ref[...]), E)
        _, gval, gidx = _topk_from_col(glog_col, E, TOPK)
        gidx_ref[0] = gidx * jnp.ones((TOPK, E), F32)
        gww_ref[0] = _softmax_col(gval) * jnp.ones((TOPK, E), F32)


def _k2_body(gidx_sref, wsm_sref, l2_ref, expw_ref, expb_ref,
             selw_ref, ww_ref, kpo_ref, vpo_ref, ll8_ref,
             okvg_ref, okvb_ref, wko_ref, bko_ref, wvo_ref, bvo_ref,
             oqg_ref, oqb_ref, wqo_ref, bqo_ref, m1_ref,
             oow_ref, oob_ref, ofg_ref, ofb_ref,
             w1_ref, b1_ref, w2_ref, b2_ref, pw_ref, pb_ref,
             state_ref, y_ref, acc_s):
    S, KW, HH, TOPK = 8, 2, 16, 2
    b = pl.program_id(0)
    k = pl.program_id(1)

    @pl.when(k == 0)
    def _():
        acc_s[...] = l2_ref[0]

    wk = wsm_sref[b * TOPK + k]
    yk = jax.nn.relu(_dot(l2_ref[0], expw_ref[0]) + expb_ref[0])
    acc_s[...] += wk * yk

    @pl.when(k == TOPK - 1)
    def _fin():
        l3 = acc_s[...]
        mean_row = jnp.mean(l3, axis=0, keepdims=True)
        ww_col = jnp.mean(ww_ref[0], axis=1, keepdims=True)
        ww_row = _to_row(ww_col, KW)
        c_row = _dx(ww_row, selw_ref[0])
        c_col = _to_col(c_row, S)
        state = _dx(c_col, mean_row)
        state_ref[0] = state
        lnst = _ln(state, okvg_ref[...], okvb_ref[...])
        kpst = _dotb(lnst, wko_ref) + bko_ref[...]
        vpst = _dotb(lnst, wvo_ref) + bvo_ref[...]
        ll = ll8_ref[0][7:8, :]
        q2 = _dotb(_ln(ll, oqg_ref[...], oqb_ref[...]), wqo_ref) \
            + bqo_ref[...]
        qbo = _dx(jnp.ones((HH, 1), F32), q2) * m1_ref[...]
        s_main = jax.lax.dot_general(qbo.astype(BF16), kpo_ref[0],
                                     DIMS_NT,
                                     preferred_element_type=F32) * 0.125
        s_st = jax.lax.dot_general(qbo, kpst, DIMS_NT,
                                   preferred_element_type=F32) * 0.125
        m = jnp.maximum(jnp.max(s_main, axis=1, keepdims=True),
                        jnp.max(s_st, axis=1, keepdims=True))
        e_main = jnp.exp(s_main - m)
        e_st = jnp.exp(s_st - m)
        l = jnp.sum(e_main, axis=1, keepdims=True) \
            + jnp.sum(e_st, axis=1, keepdims=True)
        attn = (jnp.dot(e_main.astype(BF16), vpo_ref[0],
                        preferred_element_type=F32)
                + e_st @ vpst) / l
        o = _dx(jnp.ones((1, HH), F32), m1_ref[...] * attn)
        l1 = ll + _dotb(o, oow_ref) + oob_ref[...]
        hn = _ln(l1, ofg_ref[...], ofb_ref[...])
        g = _gelu(_dotb(hn, w1_ref) + b1_ref[...])
        lf = l1 + _dotb(g, w2_ref) + b2_ref[...]
        y_ref[0] = _dotb(lf, pw_ref) + pb_ref[...]


def _full_spec(shape):
    return pl.BlockSpec(shape, lambda *a: tuple(0 for _ in shape))


def kernel(x, state_flat, params):
    p = params
    B, T, DIN = x.shape
    D = p['se_w'].shape[0]
    S = state_flat.shape[1] // D
    E = p['gate_w'].shape[0]
    H, TOPK, KR, KW = 16, 2, 4, 2
    TILE = 512
    n_t = T // TILE

    f32 = jnp.float32
    latent = state_flat.reshape(B, S, D)

    f = float(S) ** (jnp.arange(D // 2).astype(f32) / (D // 2))
    inv_f = (1.0 / f).reshape(1, -1)

    def row(v):
        return v.reshape(1, -1).astype(f32)

    sew_t = p['se_w'].T
    oew_t = p['oe_w'].T
    wq_s_t = p['smha_in_w'][:D].T
    wk_s_t = p['smha_in_w'][D:2 * D].T
    wv_s_t = p['smha_in_w'][2 * D:].T
    bq_s, bk_s, bv_s = (row(p['smha_in_b'][i * D:(i + 1) * D]) for i in range(3))
    wq_o_t = p['omha_in_w'][:D].T
    wk_o_t = p['omha_in_w'][D:2 * D].T
    wv_o_t = p['omha_in_w'][2 * D:].T
    bq_o, bk_o, bv_o = (row(p['omha_in_b'][i * D:(i + 1) * D]) for i in range(3))
    exp_b3 = p['exp_b'][:, None, :]
    bf16 = jnp.bfloat16
    wk_s_b = wk_s_t.astype(bf16)
    wv_s_b = wv_s_t.astype(bf16)
    wk_o_b = wk_o_t.astype(bf16)
    wv_o_b = wv_o_t.astype(bf16)

    DH = D // H
    r64 = jnp.arange(H * KR)
    c = jnp.arange(D)
    P4 = (r64[:, None] % KR == jnp.arange(KR)[None, :]).astype(f32)
    M4 = ((c[None, :] // DH) == (r64[:, None] // KR)).astype(f32)
    P4T = P4.T
    M1 = ((c[None, :] // DH) == jnp.arange(H)[:, None]).astype(f32)

    def bspec(rows, cols):
        return pl.BlockSpec((1, rows, cols), lambda b, *a: (b, 0, 0))

    k1_out = pl.pallas_call(
        functools.partial(_k1_body, n_t=n_t, tile=TILE),
        grid=(B, n_t),
        in_specs=[pl.BlockSpec((1, TILE, DIN), lambda b, t: (b, t, 0)),
                  _full_spec((1, D // 2)),
                  bspec(S, D), _full_spec((D, 1)), _full_spec((D, 1)),
                  _full_spec((1, D)), _full_spec((1, D)),
                  _full_spec((1, D)), _full_spec((1, D)),
                  _full_spec((1, D)), _full_spec((1, D)),
                  _full_spec((D, D)), _full_spec((1, D)),
                  _full_spec((D, D)), _full_spec((1, D)),
                  _full_spec((D, D)), _full_spec((1, D)),
                  _full_spec((H * KR, KR)), _full_spec((H * KR, D)),
                  _full_spec((KR, H * KR)),
                  _full_spec((DIN, D)), _full_spec((1, D)),
                  _full_spec((DIN, D)), _full_spec((1, D)),
                  _full_spec((1, D)), _full_spec((1, D)),
                  _full_spec((D, D)), _full_spec((1, D)),
                  _full_spec((D, D)), _full_spec((1, D)),
                  _full_spec((D, D)), _full_spec((1, D)),
                  _full_spec((1, D)), _full_spec((1, D)),
                  _full_spec((D, D)), _full_spec((1, D)),
                  _full_spec((D, D)), _full_spec((1, D)),
                  _full_spec((1, D)), _full_spec((1, D)),
                  _full_spec((D, E))],
        out_specs=[bspec(KR, S), bspec(KW, S), bspec(KW, S), bspec(KW, S),
                   pl.BlockSpec((1, TILE, D), lambda b, t: (b, t, 0)),
                   pl.BlockSpec((1, TILE, D), lambda b, t: (b, t, 0)),
                   bspec(8, D), bspec(KR, D), bspec(TOPK, E), bspec(TOPK, E)],
        out_shape=[jax.ShapeDtypeStruct((B, KR, S), f32),
                   jax.ShapeDtypeStruct((B, KW, S), f32),
                   jax.ShapeDtypeStruct((B, KW, S), f32),
                   jax.ShapeDtypeStruct((B, KW, S), f32),
                   jax.ShapeDtypeStruct((B, T, D), jnp.bfloat16),
                   jax.ShapeDtypeStruct((B, T, D), jnp.bfloat16),
                   jax.ShapeDtypeStruct((B, 8, D), f32),
                   jax.ShapeDtypeStruct((B, KR, D), f32),
                   jax.ShapeDtypeStruct((B, TOPK, E), f32),
                   jax.ShapeDtypeStruct((B, TOPK, E), f32)],
        scratch_shapes=[pltpu.VMEM((H * KR, D), f32),
                        pltpu.VMEM((KR, D), f32),
                        pltpu.VMEM((KR, D), f32),
                        pltpu.VMEM((KR, D), f32),
                        pltpu.VMEM((H * KR, 128), f32),
                        pltpu.VMEM((H * KR, 128), f32),
                        pltpu.VMEM((H * KR, D), f32)],
        interpret=INTERP,
    )(x, inv_f, latent, p['read_w'].T, p['slot_w'].T,
      row(p['sln_slot_g']), row(p['sln_slot_b']),
      row(p['sln_q_g']), row(p['sln_q_b']),
      row(p['sln_kv_g']), row(p['sln_kv_b']),
      wq_s_t.astype(bf16), bq_s, wk_s_b, bk_s, wv_s_b, bv_s,
      P4, M4, P4T,
      sew_t.astype(bf16), row(p['se_b']), oew_t.astype(bf16), row(p['oe_b']),
      row(p['oln_kv_g']), row(p['oln_kv_b']),
      wk_o_b, bk_o, wv_o_b, bv_o,
      p['smha_out_w'].T.astype(bf16), row(p['smha_out_b']),
      row(p['sln_ffn_g']), row(p['sln_ffn_b']),
      p['sffn_w1'].T.astype(bf16), row(p['sffn_b1']),
      p['sffn_w2'].T.astype(bf16), row(p['sffn_b2']),
      row(p['sln_moe_g']), row(p['sln_moe_b']), p['gate_w'].T)

    (ridx_b, widx_b, ww_b, selw, kpo, vpo, ll8, l2,
     gidx_b, gw_b) = k1_out

    gidx = gidx_b[:, :, 0].astype(jnp.int32)
    gidx_flat = gidx.reshape(-1)
    w_flat = gw_b[:, :, 0].reshape(-1)

    grid_spec = pltpu.PrefetchScalarGridSpec(
        num_scalar_prefetch=2,
        grid=(B, TOPK),
        in_specs=[pl.BlockSpec((1, KR, D), lambda b, k, gref, wref: (b, 0, 0)),
                  pl.BlockSpec((1, D, D),
                               lambda b, k, gref, wref: (gref[b * 2 + k], 0, 0)),
                  pl.BlockSpec((1, 1, D),
                               lambda b, k, gref, wref: (gref[b * 2 + k], 0, 0)),
                  pl.BlockSpec((1, KW, S), lambda b, k, gref, wref: (b, 0, 0)),
                  pl.BlockSpec((1, KW, S), lambda b, k, gref, wref: (b, 0, 0)),
                  pl.BlockSpec((1, T, D), lambda b, k, gref, wref: (b, 0, 0)),
                  pl.BlockSpec((1, T, D), lambda b, k, gref, wref: (b, 0, 0)),
                  pl.BlockSpec((1, 8, D), lambda b, k, gref, wref: (b, 0, 0))]
        + [pl.BlockSpec(s, lambda b, k, gref, wref, _s=s:
                        tuple(0 for _ in _s))
           for s in [(1, D), (1, D), (D, D), (1, D), (D, D), (1, D),
                     (1, D), (1, D), (D, D), (1, D), (H, D),
                     (D, D), (1, D), (1, D), (1, D),
                     (D, D), (1, D), (D, D), (1, D),
                     (D, p['outp_w'].shape[0]), (1, p['outp_w'].shape[0])]],
        out_specs=[pl.BlockSpec((1, S, D), lambda b, k, gref, wref: (b, 0, 0)),
                   pl.BlockSpec((1, 1, p['outp_w'].shape[0]),
                                lambda b, k, gref, wref: (b, 0, 0))],
        scratch_shapes=[pltpu.VMEM((KR, D), f32)],
    )
    state3, y3 = pl.pallas_call(
        _k2_body,
        grid_spec=grid_spec,
        out_shape=[jax.ShapeDtypeStruct((B, S, D), f32),
                   jax.ShapeDtypeStruct((B, 1, p['outp_w'].shape[0]), f32)],
        interpret=INTERP,
    )(gidx_flat, w_flat, l2, p['exp_w'], exp_b3,
      selw, ww_b, kpo, vpo, ll8,
      row(p['oln_kv_g']), row(p['oln_kv_b']),
      wk_o_b, bk_o, wv_o_b, bv_o,
      row(p['oln_q_g']), row(p['oln_q_b']), wq_o_t.astype(bf16), bq_o, M1,
      p['omha_out_w'].T.astype(bf16), row(p['omha_out_b']),
      row(p['oln_ffn_g']), row(p['oln_ffn_b']),
      p['offn_w1'].T.astype(bf16), row(p['offn_b1']),
      p['offn_w2'].T.astype(bf16), row(p['offn_b2']),
      p['outp_w'].T.astype(bf16), row(p['outp_b']))

    y = y3[:, 0, :]
    read_idx = ridx_b[:, :, 0].astype(jnp.int32)
    write_idx = widx_b[:, :, 0].astype(jnp.int32)
    state_out = state3.reshape(B, S * D)
    return y, gidx, read_idx, write_idx, state_out
